# baseline (speedup 1.0000x reference)
import functools

import jax
import jax.numpy as jnp
from jax import lax
from jax.experimental import pallas as pl
from jax.experimental.pallas import tpu as pltpu
from jax.experimental.pallas import tpu_sc as plsc

_NC = 2
_NS = 16
_NW = _NC * _NS
_L = 16
_J = 10
_TW = 128
_NBUF = 3
_SC_ROWS = 10240
_BR = 512


@functools.cache
def _make_sc_kernel(B, N, S):
    b_per_w = S // _NW
    bc = b_per_w // 2
    n_chunks = b_per_w // bc
    grp_per_chunk = bc // _L

    mesh = plsc.VectorSubcoreMesh(
        core_axis_name="c", subcore_axis_name="s",
        num_cores=_NC, num_subcores=_NS)

    @functools.partial(
        pl.kernel,
        out_type=jax.ShapeDtypeStruct((S,), jnp.float32),
        mesh=mesh,
        compiler_params=pltpu.CompilerParams(needs_layout_passes=False),
        scratch_types=[
            pltpu.VMEM((_L,), jnp.float32),
            pltpu.VMEM((_L,), jnp.float32),
            pltpu.VMEM((_L,), jnp.float32),
            pltpu.VMEM((_L,), jnp.int32),
            pltpu.VMEM((_NBUF, bc, _TW), jnp.float32),
            pltpu.VMEM((b_per_w,), jnp.float32),
            pltpu.SemaphoreType.DMA,
            pltpu.SemaphoreType.DMA,
            pltpu.SemaphoreType.DMA,
        ],
    )
    def sc_kernel(x2d, mean_in, std_in, vals_in, idx_in, out,
                  mean_v, std_v, vals_v, idx_v, ring, outbuf,
                  sem0, sem1, sem2):
        sems = (sem0, sem1, sem2)
        wid = lax.axis_index("s") * _NC + lax.axis_index("c")
        base = wid * b_per_w

        cps = [
            pltpu.async_copy(mean_in, mean_v.at[pl.ds(0, _J)], sem0),
            pltpu.async_copy(std_in, std_v.at[pl.ds(0, _J)], sem0),
            pltpu.async_copy(vals_in, vals_v.at[pl.ds(0, _J)], sem0),
            pltpu.async_copy(idx_in, idx_v.at[pl.ds(0, _J)], sem0),
        ]
        for cp in cps:
            cp.wait()

        lane = lax.iota(jnp.int32, _L)
        inb = lane < _J

        def lane_scalar(vec, j):
            return jnp.sum(jnp.where(lane == j, vec, jnp.zeros_like(vec)))

        mean_v16 = mean_v[...]
        std_v16 = std_v[...]
        vals_v16 = vals_v[...]
        iv = idx_v[...]

        col = [lane_scalar(iv, j) for j in range(_J)]
        band = [(col[j] // _TW) * _TW for j in range(_J)]
        coff = [jnp.full((_L,), col[j] % _TW) for j in range(_J)]

        wv = vals_v16 / std_v16
        c = -jnp.sum(jnp.where(inb, vals_v16 * mean_v16 / std_v16, 0.0))
        wj = [lane_scalar(wv, j) for j in range(_J)]

        work = [(ch, j) for ch in range(n_chunks) for j in range(_J)]

        def fire(k):
            ch, j = work[k]
            slot = k % _NBUF
            return pltpu.async_copy(
                x2d.at[pl.ds(base + ch * bc, bc), pl.ds(band[j], _TW)],
                ring.at[slot], sems[slot])

        copies = {}
        for k in range(_NBUF):
            copies[k] = fire(k)

        rows = [jnp.full((_L,), g * _L) + lane for g in range(grp_per_chunk)]
        for ch in range(n_chunks):
            acc = [jnp.full((_L,), c, jnp.float32)
                   for _ in range(grp_per_chunk)]
            for j in range(_J):
                k = ch * _J + j
                slot = k % _NBUF
                copies[k].wait()
                for g in range(grp_per_chunk):
                    vec = plsc.load_gather(ring.at[slot], [rows[g], coff[j]])
                    acc[g] = acc[g] + wj[j] * vec
                nxt = k + _NBUF
                if nxt < len(work):
                    copies[nxt] = fire(nxt)
            for g in range(grp_per_chunk):
                outbuf[pl.ds(ch * bc + g * _L, _L)] = acc[g]

        pltpu.sync_copy(outbuf, out.at[pl.ds(base, b_per_w)])

    return sc_kernel


@functools.cache
def _make_tc_kernel(B, N, S):
    R = B - S
    gi = R // _BR

    def x_map(i, j, bt_ref):
        return (S // _BR + i, bt_ref[j])

    def o_map(i, j, bt_ref):
        return (i,)

    grid_spec = pltpu.PrefetchScalarGridSpec(
        num_scalar_prefetch=1,
        grid=(gi, _J),
        in_specs=[
            pl.BlockSpec((_BR, _TW), x_map),
            pl.BlockSpec(memory_space=pltpu.SMEM),
            pl.BlockSpec(memory_space=pltpu.SMEM),
            pl.BlockSpec(memory_space=pltpu.SMEM),
        ],
        out_specs=pl.BlockSpec((_BR,), o_map),
    )

    def tc_body(bt_ref, x_ref, w_ref, coff_ref, c_ref, o_ref):
        j = pl.program_id(1)
        blk = x_ref[...]
        cmask = lax.broadcasted_iota(jnp.int32, (_BR, _TW), 1) == coff_ref[j]
        contrib = jnp.sum(jnp.where(cmask, blk, 0.0), axis=1) * w_ref[j]

        @pl.when(j == 0)
        def _():
            o_ref[...] = contrib + c_ref[0]

        @pl.when(j != 0)
        def _():
            o_ref[...] = o_ref[...] + contrib

    return pl.pallas_call(
        tc_body,
        grid_spec=grid_spec,
        out_shape=jax.ShapeDtypeStruct((R,), jnp.float32),
        compiler_params=pltpu.CompilerParams(
            dimension_semantics=("parallel", "arbitrary")),
    )


def kernel(X_neuron, mean, std, vals, idx):
    B, N = X_neuron.shape
    out_sc = _make_sc_kernel(B, N, _SC_ROWS)(X_neuron, mean, std, vals, idx)
    w = vals / std
    c = -jnp.sum(vals * mean / std)
    bt = idx // _TW
    coff = idx % _TW
    out_tc = _make_tc_kernel(B, N, _SC_ROWS)(
        bt, X_neuron, w, coff, c[None])
    return jnp.concatenate([out_sc, out_tc])


# speedup vs baseline: 1.3814x; 1.3814x over previous
import functools

import jax
import jax.numpy as jnp
from jax import lax
from jax.experimental import pallas as pl
from jax.experimental.pallas import tpu as pltpu
from jax.experimental.pallas import tpu_sc as plsc

_NC = 2
_NS = 16
_NW = _NC * _NS
_L = 16
_J = 10
_TW = 128
_NBUF = 3
_SC_ROWS = 10240
_BR = 1024


@functools.cache
def _make_sc_kernel(B, N, S):
    b_per_w = S // _NW
    bc = b_per_w // 2
    n_chunks = b_per_w // bc
    grp_per_chunk = bc // _L

    mesh = plsc.VectorSubcoreMesh(
        core_axis_name="c", subcore_axis_name="s",
        num_cores=_NC, num_subcores=_NS)

    @functools.partial(
        pl.kernel,
        out_type=jax.ShapeDtypeStruct((S,), jnp.float32),
        mesh=mesh,
        compiler_params=pltpu.CompilerParams(needs_layout_passes=False),
        scratch_types=[
            pltpu.VMEM((_L,), jnp.float32),
            pltpu.VMEM((_L,), jnp.float32),
            pltpu.VMEM((_L,), jnp.float32),
            pltpu.VMEM((_L,), jnp.int32),
            pltpu.VMEM((_NBUF, bc, _TW), jnp.float32),
            pltpu.VMEM((b_per_w,), jnp.float32),
            pltpu.SemaphoreType.DMA,
            pltpu.SemaphoreType.DMA,
            pltpu.SemaphoreType.DMA,
        ],
    )
    def sc_kernel(x2d, mean_in, std_in, vals_in, idx_in, out,
                  mean_v, std_v, vals_v, idx_v, ring, outbuf,
                  sem0, sem1, sem2):
        sems = (sem0, sem1, sem2)
        wid = lax.axis_index("s") * _NC + lax.axis_index("c")
        base = wid * b_per_w

        cps = [
            pltpu.async_copy(mean_in, mean_v.at[pl.ds(0, _J)], sem0),
            pltpu.async_copy(std_in, std_v.at[pl.ds(0, _J)], sem0),
            pltpu.async_copy(vals_in, vals_v.at[pl.ds(0, _J)], sem0),
            pltpu.async_copy(idx_in, idx_v.at[pl.ds(0, _J)], sem0),
        ]
        for cp in cps:
            cp.wait()

        lane = lax.iota(jnp.int32, _L)
        inb = lane < _J

        def lane_scalar(vec, j):
            return jnp.sum(jnp.where(lane == j, vec, jnp.zeros_like(vec)))

        mean_v16 = mean_v[...]
        std_v16 = std_v[...]
        vals_v16 = vals_v[...]
        iv = idx_v[...]

        col = [lane_scalar(iv, j) for j in range(_J)]
        band = [(col[j] // _TW) * _TW for j in range(_J)]
        coff = [jnp.full((_L,), col[j] % _TW) for j in range(_J)]

        wv = vals_v16 / std_v16
        c = -jnp.sum(jnp.where(inb, vals_v16 * mean_v16 / std_v16, 0.0))
        wj = [lane_scalar(wv, j) for j in range(_J)]

        work = [(ch, j) for ch in range(n_chunks) for j in range(_J)]

        def fire(k):
            ch, j = work[k]
            slot = k % _NBUF
            return pltpu.async_copy(
                x2d.at[pl.ds(base + ch * bc, bc), pl.ds(band[j], _TW)],
                ring.at[slot], sems[slot])

        copies = {}
        for k in range(_NBUF):
            copies[k] = fire(k)

        rows = [jnp.full((_L,), g * _L) + lane for g in range(grp_per_chunk)]
        for ch in range(n_chunks):
            acc = [jnp.full((_L,), c, jnp.float32)
                   for _ in range(grp_per_chunk)]
            for j in range(_J):
                k = ch * _J + j
                slot = k % _NBUF
                copies[k].wait()
                for g in range(grp_per_chunk):
                    vec = plsc.load_gather(ring.at[slot], [rows[g], coff[j]])
                    acc[g] = acc[g] + wj[j] * vec
                nxt = k + _NBUF
                if nxt < len(work):
                    copies[nxt] = fire(nxt)
            for g in range(grp_per_chunk):
                outbuf[pl.ds(ch * bc + g * _L, _L)] = acc[g]

        pltpu.sync_copy(outbuf, out.at[pl.ds(base, b_per_w)])

    return sc_kernel


@functools.cache
def _make_tc_kernel(B, N, S):
    R = B - S
    gi = R // _BR

    def x_map(i, j, bt_ref):
        return (S // _BR + i, bt_ref[j])

    def o_map(i, j, bt_ref):
        return (i,)

    grid_spec = pltpu.PrefetchScalarGridSpec(
        num_scalar_prefetch=1,
        grid=(gi, _J),
        in_specs=[
            pl.BlockSpec((_BR, _TW), x_map),
            pl.BlockSpec(memory_space=pltpu.SMEM),
            pl.BlockSpec(memory_space=pltpu.SMEM),
            pl.BlockSpec(memory_space=pltpu.SMEM),
        ],
        out_specs=pl.BlockSpec((_BR,), o_map),
    )

    def tc_body(bt_ref, x_ref, w_ref, coff_ref, c_ref, o_ref):
        j = pl.program_id(1)
        blk = x_ref[...]
        onehot = jnp.where(
            lax.broadcasted_iota(jnp.int32, (_TW, 1), 0) == coff_ref[j],
            w_ref[j], 0.0)
        contrib = jnp.dot(blk, onehot,
                          preferred_element_type=jnp.float32)[:, 0]

        @pl.when(j == 0)
        def _():
            o_ref[...] = contrib + c_ref[0]

        @pl.when(j != 0)
        def _():
            o_ref[...] = o_ref[...] + contrib

    return pl.pallas_call(
        tc_body,
        grid_spec=grid_spec,
        out_shape=jax.ShapeDtypeStruct((R,), jnp.float32),
        compiler_params=pltpu.CompilerParams(
            dimension_semantics=("parallel", "arbitrary")),
    )


def kernel(X_neuron, mean, std, vals, idx):
    B, N = X_neuron.shape
    out_sc = _make_sc_kernel(B, N, _SC_ROWS)(X_neuron, mean, std, vals, idx)
    w = vals / std
    c = -jnp.sum(vals * mean / std)
    bt = idx // _TW
    coff = idx % _TW
    out_tc = _make_tc_kernel(B, N, _SC_ROWS)(
        bt, X_neuron, w, coff, c[None])
    return jnp.concatenate([out_sc, out_tc])


# speedup vs baseline: 1.8658x; 1.3506x over previous
import functools

import jax
import jax.numpy as jnp
from jax import lax
from jax.experimental import pallas as pl
from jax.experimental.pallas import tpu as pltpu
from jax.experimental.pallas import tpu_sc as plsc

_NC = 2
_NS = 16
_NW = _NC * _NS
_L = 16
_J = 10
_TW = 128
_NBUF = 3
_SC_ROWS = 13312
_BR = 1024


@functools.cache
def _make_sc_kernel(B, N, S):
    b_per_w = S // _NW
    bc = b_per_w // 2
    n_chunks = b_per_w // bc
    grp_per_chunk = bc // _L

    mesh = plsc.VectorSubcoreMesh(
        core_axis_name="c", subcore_axis_name="s",
        num_cores=_NC, num_subcores=_NS)

    @functools.partial(
        pl.kernel,
        out_type=jax.ShapeDtypeStruct((S,), jnp.float32),
        mesh=mesh,
        compiler_params=pltpu.CompilerParams(needs_layout_passes=False),
        scratch_types=[
            pltpu.VMEM((_L,), jnp.float32),
            pltpu.VMEM((_L,), jnp.float32),
            pltpu.VMEM((_L,), jnp.float32),
            pltpu.VMEM((_L,), jnp.int32),
            pltpu.VMEM((_NBUF, bc, _TW), jnp.float32),
            pltpu.VMEM((b_per_w,), jnp.float32),
            pltpu.SemaphoreType.DMA,
            pltpu.SemaphoreType.DMA,
            pltpu.SemaphoreType.DMA,
        ],
    )
    def sc_kernel(x2d, mean_in, std_in, vals_in, idx_in, out,
                  mean_v, std_v, vals_v, idx_v, ring, outbuf,
                  sem0, sem1, sem2):
        sems = (sem0, sem1, sem2)
        wid = lax.axis_index("s") * _NC + lax.axis_index("c")
        base = wid * b_per_w

        cps = [
            pltpu.async_copy(mean_in, mean_v.at[pl.ds(0, _J)], sem0),
            pltpu.async_copy(std_in, std_v.at[pl.ds(0, _J)], sem0),
            pltpu.async_copy(vals_in, vals_v.at[pl.ds(0, _J)], sem0),
            pltpu.async_copy(idx_in, idx_v.at[pl.ds(0, _J)], sem0),
        ]
        for cp in cps:
            cp.wait()

        lane = lax.iota(jnp.int32, _L)
        inb = lane < _J

        def lane_scalar(vec, j):
            return jnp.sum(jnp.where(lane == j, vec, jnp.zeros_like(vec)))

        mean_v16 = mean_v[...]
        std_v16 = std_v[...]
        vals_v16 = vals_v[...]
        iv = idx_v[...]

        col = [lane_scalar(iv, j) for j in range(_J)]
        band = [(col[j] // _TW) * _TW for j in range(_J)]
        coff = [jnp.full((_L,), col[j] % _TW) for j in range(_J)]

        wv = vals_v16 / std_v16
        c = -jnp.sum(jnp.where(inb, vals_v16 * mean_v16 / std_v16, 0.0))
        wj = [lane_scalar(wv, j) for j in range(_J)]

        work = [(ch, j) for ch in range(n_chunks) for j in range(_J)]

        def fire(k):
            ch, j = work[k]
            slot = k % _NBUF
            return pltpu.async_copy(
                x2d.at[pl.ds(base + ch * bc, bc), pl.ds(band[j], _TW)],
                ring.at[slot], sems[slot])

        copies = {}
        for k in range(_NBUF):
            copies[k] = fire(k)

        rows = [jnp.full((_L,), g * _L) + lane for g in range(grp_per_chunk)]
        for ch in range(n_chunks):
            acc = [jnp.full((_L,), c, jnp.float32)
                   for _ in range(grp_per_chunk)]
            for j in range(_J):
                k = ch * _J + j
                slot = k % _NBUF
                copies[k].wait()
                for g in range(grp_per_chunk):
                    vec = plsc.load_gather(ring.at[slot], [rows[g], coff[j]])
                    acc[g] = acc[g] + wj[j] * vec
                nxt = k + _NBUF
                if nxt < len(work):
                    copies[nxt] = fire(nxt)
            for g in range(grp_per_chunk):
                outbuf[pl.ds(ch * bc + g * _L, _L)] = acc[g]

        pltpu.sync_copy(outbuf, out.at[pl.ds(base, b_per_w)])

    return sc_kernel


@functools.cache
def _make_tc_kernel(B, N, S):
    R = B - S
    gi = R // _BR

    def x_map(i, j, bt_ref):
        return (S // _BR + i, bt_ref[j])

    def o_map(i, j, bt_ref):
        return (i,)

    grid_spec = pltpu.PrefetchScalarGridSpec(
        num_scalar_prefetch=1,
        grid=(gi, _J),
        in_specs=[
            pl.BlockSpec((_BR, _TW), x_map),
            pl.BlockSpec(memory_space=pltpu.SMEM),
            pl.BlockSpec(memory_space=pltpu.SMEM),
            pl.BlockSpec(memory_space=pltpu.SMEM),
        ],
        out_specs=pl.BlockSpec((_BR,), o_map),
    )

    def tc_body(bt_ref, x_ref, w_ref, coff_ref, c_ref, o_ref):
        j = pl.program_id(1)
        blk = x_ref[...]
        onehot = jnp.where(
            lax.broadcasted_iota(jnp.int32, (_TW, 1), 0) == coff_ref[j],
            w_ref[j], 0.0)
        contrib = jnp.dot(blk, onehot, precision=lax.Precision.HIGHEST,
                          preferred_element_type=jnp.float32)[:, 0]

        @pl.when(j == 0)
        def _():
            o_ref[...] = contrib + c_ref[0]

        @pl.when(j != 0)
        def _():
            o_ref[...] = o_ref[...] + contrib

    return pl.pallas_call(
        tc_body,
        grid_spec=grid_spec,
        out_shape=jax.ShapeDtypeStruct((R,), jnp.float32),
        compiler_params=pltpu.CompilerParams(
            dimension_semantics=("parallel", "arbitrary")),
    )


def kernel(X_neuron, mean, std, vals, idx):
    B, N = X_neuron.shape
    out_sc = _make_sc_kernel(B, N, _SC_ROWS)(X_neuron, mean, std, vals, idx)
    w = vals / std
    c = -jnp.sum(vals * mean / std)
    bt = idx // _TW
    coff = idx % _TW
    out_tc = _make_tc_kernel(B, N, _SC_ROWS)(
        bt, X_neuron, w, coff, c[None])
    return jnp.concatenate([out_sc, out_tc])


# speedup vs baseline: 2.0877x; 1.1189x over previous
import functools

import jax
import jax.numpy as jnp
from jax import lax
from jax.experimental import pallas as pl
from jax.experimental.pallas import tpu as pltpu
from jax.experimental.pallas import tpu_sc as plsc

_NC = 2
_NS = 16
_NW = _NC * _NS
_L = 16
_J = 10
_TW = 128
_NBUF = 3
_SC_ROWS = 12288
_BR = 1024


@functools.cache
def _make_sc_kernel(B, N, S):
    b_per_w = S // _NW
    bc = b_per_w // 2
    n_chunks = b_per_w // bc
    grp_per_chunk = bc // _L

    mesh = plsc.VectorSubcoreMesh(
        core_axis_name="c", subcore_axis_name="s",
        num_cores=_NC, num_subcores=_NS)

    @functools.partial(
        pl.kernel,
        out_type=jax.ShapeDtypeStruct((S,), jnp.float32),
        mesh=mesh,
        compiler_params=pltpu.CompilerParams(needs_layout_passes=False),
        scratch_types=[
            pltpu.VMEM((_L,), jnp.float32),
            pltpu.VMEM((_L,), jnp.float32),
            pltpu.VMEM((_L,), jnp.float32),
            pltpu.VMEM((_L,), jnp.int32),
            pltpu.VMEM((_NBUF, bc, _TW), jnp.float32),
            pltpu.VMEM((b_per_w,), jnp.float32),
            pltpu.SemaphoreType.DMA,
            pltpu.SemaphoreType.DMA,
            pltpu.SemaphoreType.DMA,
        ],
    )
    def sc_kernel(x2d, mean_in, std_in, vals_in, idx_in, out,
                  mean_v, std_v, vals_v, idx_v, ring, outbuf,
                  sem0, sem1, sem2):
        sems = (sem0, sem1, sem2)
        wid = lax.axis_index("s") * _NC + lax.axis_index("c")
        base = wid * b_per_w

        cps = [
            pltpu.async_copy(mean_in, mean_v.at[pl.ds(0, _J)], sem0),
            pltpu.async_copy(std_in, std_v.at[pl.ds(0, _J)], sem0),
            pltpu.async_copy(vals_in, vals_v.at[pl.ds(0, _J)], sem0),
            pltpu.async_copy(idx_in, idx_v.at[pl.ds(0, _J)], sem0),
        ]
        for cp in cps:
            cp.wait()

        lane = lax.iota(jnp.int32, _L)
        inb = lane < _J

        def lane_scalar(vec, j):
            return jnp.sum(jnp.where(lane == j, vec, jnp.zeros_like(vec)))

        mean_v16 = mean_v[...]
        std_v16 = std_v[...]
        vals_v16 = vals_v[...]
        iv = idx_v[...]

        col = [lane_scalar(iv, j) for j in range(_J)]
        band = [(col[j] // _TW) * _TW for j in range(_J)]
        coff = [jnp.full((_L,), col[j] % _TW) for j in range(_J)]

        wv = vals_v16 / std_v16
        c = -jnp.sum(jnp.where(inb, vals_v16 * mean_v16 / std_v16, 0.0))
        wj = [lane_scalar(wv, j) for j in range(_J)]

        work = [(ch, j) for ch in range(n_chunks) for j in range(_J)]

        def fire(k):
            ch, j = work[k]
            slot = k % _NBUF
            return pltpu.async_copy(
                x2d.at[pl.ds(base + ch * bc, bc), pl.ds(band[j], _TW)],
                ring.at[slot], sems[slot])

        copies = {}
        for k in range(_NBUF):
            copies[k] = fire(k)

        rows = [jnp.full((_L,), g * _L) + lane for g in range(grp_per_chunk)]
        for ch in range(n_chunks):
            acc = [jnp.full((_L,), c, jnp.float32)
                   for _ in range(grp_per_chunk)]
            for j in range(_J):
                k = ch * _J + j
                slot = k % _NBUF
                copies[k].wait()
                for g in range(grp_per_chunk):
                    vec = plsc.load_gather(ring.at[slot], [rows[g], coff[j]])
                    acc[g] = acc[g] + wj[j] * vec
                nxt = k + _NBUF
                if nxt < len(work):
                    copies[nxt] = fire(nxt)
            for g in range(grp_per_chunk):
                outbuf[pl.ds(ch * bc + g * _L, _L)] = acc[g]

        pltpu.sync_copy(outbuf, out.at[pl.ds(base, b_per_w)])

    return sc_kernel


@functools.cache
def _make_tc_kernel(B, N, S):
    R = B - S

    def tc_body(bt_ref, x_ref, w_ref, coff_ref, c_ref, o_ref, bufs, sems):
        cps = []
        for j in range(_J):
            cstart = pl.multiple_of(bt_ref[j] * _TW, _TW)
            cp = pltpu.make_async_copy(
                x_ref.at[pl.ds(S, R), pl.ds(cstart, _TW)],
                bufs.at[j], sems.at[j])
            cp.start()
            cps.append(cp)
        acc = jnp.full((R,), c_ref[0], jnp.float32)
        iota = lax.broadcasted_iota(jnp.int32, (1, _TW), 1)
        for j in range(_J):
            cps[j].wait()
            onehot = jnp.where(iota == coff_ref[j], w_ref[j], 0.0)[0]
            acc = acc + jnp.dot(bufs[j], onehot,
                                precision=lax.Precision.HIGHEST,
                                preferred_element_type=jnp.float32)
        o_ref[...] = acc

    grid_spec = pltpu.PrefetchScalarGridSpec(
        num_scalar_prefetch=1,
        grid=(1,),
        in_specs=[
            pl.BlockSpec(memory_space=pltpu.MemorySpace.HBM),
            pl.BlockSpec(memory_space=pltpu.SMEM),
            pl.BlockSpec(memory_space=pltpu.SMEM),
            pl.BlockSpec(memory_space=pltpu.SMEM),
        ],
        out_specs=pl.BlockSpec((R,), lambda i, bt: (0,)),
        scratch_shapes=[
            pltpu.VMEM((_J, R, _TW), jnp.float32),
            pltpu.SemaphoreType.DMA((_J,)),
        ],
    )

    return pl.pallas_call(
        tc_body,
        grid_spec=grid_spec,
        out_shape=jax.ShapeDtypeStruct((R,), jnp.float32),
    )


def kernel(X_neuron, mean, std, vals, idx):
    B, N = X_neuron.shape
    out_sc = _make_sc_kernel(B, N, _SC_ROWS)(X_neuron, mean, std, vals, idx)
    w = vals / std
    c = -jnp.sum(vals * mean / std)
    bt = idx // _TW
    coff = idx % _TW
    out_tc = _make_tc_kernel(B, N, _SC_ROWS)(
        bt, X_neuron, w, coff, c[None])
    return jnp.concatenate([out_sc, out_tc])


# speedup vs baseline: 2.1446x; 1.0273x over previous
import functools

import jax
import jax.numpy as jnp
from jax import lax
from jax.experimental import pallas as pl
from jax.experimental.pallas import tpu as pltpu
from jax.experimental.pallas import tpu_sc as plsc

_NC = 2
_NS = 16
_NW = _NC * _NS
_L = 16
_J = 10
_TW = 128
_NBUF = 3
_SC_ROWS = 10240
_BR = 1024


@functools.cache
def _make_sc_kernel(B, N, S):
    b_per_w = S // _NW
    bc = b_per_w // 2
    n_chunks = b_per_w // bc
    grp_per_chunk = bc // _L

    mesh = plsc.VectorSubcoreMesh(
        core_axis_name="c", subcore_axis_name="s",
        num_cores=_NC, num_subcores=_NS)

    @functools.partial(
        pl.kernel,
        out_type=jax.ShapeDtypeStruct((S,), jnp.float32),
        mesh=mesh,
        compiler_params=pltpu.CompilerParams(needs_layout_passes=False),
        scratch_types=[
            pltpu.VMEM((_L,), jnp.float32),
            pltpu.VMEM((_L,), jnp.float32),
            pltpu.VMEM((_L,), jnp.float32),
            pltpu.VMEM((_L,), jnp.int32),
            pltpu.VMEM((_NBUF, bc, _TW), jnp.float32),
            pltpu.VMEM((b_per_w,), jnp.float32),
            pltpu.SemaphoreType.DMA,
            pltpu.SemaphoreType.DMA,
            pltpu.SemaphoreType.DMA,
        ],
    )
    def sc_kernel(x2d, mean_in, std_in, vals_in, idx_in, out,
                  mean_v, std_v, vals_v, idx_v, ring, outbuf,
                  sem0, sem1, sem2):
        sems = (sem0, sem1, sem2)
        wid = lax.axis_index("s") * _NC + lax.axis_index("c")
        base = wid * b_per_w

        cps = [
            pltpu.async_copy(mean_in, mean_v.at[pl.ds(0, _J)], sem0),
            pltpu.async_copy(std_in, std_v.at[pl.ds(0, _J)], sem0),
            pltpu.async_copy(vals_in, vals_v.at[pl.ds(0, _J)], sem0),
            pltpu.async_copy(idx_in, idx_v.at[pl.ds(0, _J)], sem0),
        ]
        for cp in cps:
            cp.wait()

        lane = lax.iota(jnp.int32, _L)
        inb = lane < _J

        def lane_scalar(vec, j):
            return jnp.sum(jnp.where(lane == j, vec, jnp.zeros_like(vec)))

        mean_v16 = mean_v[...]
        std_v16 = std_v[...]
        vals_v16 = vals_v[...]
        iv = idx_v[...]

        col = [lane_scalar(iv, j) for j in range(_J)]
        band = [(col[j] // _TW) * _TW for j in range(_J)]
        coff = [jnp.full((_L,), col[j] % _TW) for j in range(_J)]

        wv = vals_v16 / std_v16
        c = -jnp.sum(jnp.where(inb, vals_v16 * mean_v16 / std_v16, 0.0))
        wj = [lane_scalar(wv, j) for j in range(_J)]

        work = [(ch, j) for ch in range(n_chunks) for j in range(_J)]

        def fire(k):
            ch, j = work[k]
            slot = k % _NBUF
            return pltpu.async_copy(
                x2d.at[pl.ds(base + ch * bc, bc), pl.ds(band[j], _TW)],
                ring.at[slot], sems[slot])

        copies = {}
        for k in range(_NBUF):
            copies[k] = fire(k)

        rows = [jnp.full((_L,), g * _L) + lane for g in range(grp_per_chunk)]
        for ch in range(n_chunks):
            acc = [jnp.full((_L,), c, jnp.float32)
                   for _ in range(grp_per_chunk)]
            for j in range(_J):
                k = ch * _J + j
                slot = k % _NBUF
                copies[k].wait()
                for g in range(grp_per_chunk):
                    vec = plsc.load_gather(ring.at[slot], [rows[g], coff[j]])
                    acc[g] = acc[g] + wj[j] * vec
                nxt = k + _NBUF
                if nxt < len(work):
                    copies[nxt] = fire(nxt)
            for g in range(grp_per_chunk):
                outbuf[pl.ds(ch * bc + g * _L, _L)] = acc[g]

        pltpu.sync_copy(outbuf, out.at[pl.ds(base, b_per_w)])

    return sc_kernel


@functools.cache
def _make_tc_kernel(B, N, S):
    R = B - S

    def tc_body(bt_ref, x_ref, w_ref, coff_ref, c_ref, o_ref, bufs, sems):
        cps = []
        for j in range(_J):
            cstart = pl.multiple_of(bt_ref[j] * _TW, _TW)
            cp = pltpu.make_async_copy(
                x_ref.at[pl.ds(S, R), pl.ds(cstart, _TW)],
                bufs.at[j], sems.at[j])
            cp.start()
            cps.append(cp)
        acc = jnp.full((R,), c_ref[0], jnp.float32)
        iota = lax.broadcasted_iota(jnp.int32, (1, _TW), 1)
        for j in range(_J):
            cps[j].wait()
            onehot = jnp.where(iota == coff_ref[j], w_ref[j], 0.0)[0]
            acc = acc + jnp.dot(bufs[j], onehot,
                                precision=lax.Precision.HIGHEST,
                                preferred_element_type=jnp.float32)
        o_ref[...] = acc

    grid_spec = pltpu.PrefetchScalarGridSpec(
        num_scalar_prefetch=1,
        grid=(1,),
        in_specs=[
            pl.BlockSpec(memory_space=pltpu.MemorySpace.HBM),
            pl.BlockSpec(memory_space=pltpu.SMEM),
            pl.BlockSpec(memory_space=pltpu.SMEM),
            pl.BlockSpec(memory_space=pltpu.SMEM),
        ],
        out_specs=pl.BlockSpec((R,), lambda i, bt: (0,)),
        scratch_shapes=[
            pltpu.VMEM((_J, R, _TW), jnp.float32),
            pltpu.SemaphoreType.DMA((_J,)),
        ],
    )

    return pl.pallas_call(
        tc_body,
        grid_spec=grid_spec,
        out_shape=jax.ShapeDtypeStruct((R,), jnp.float32),
    )


def kernel(X_neuron, mean, std, vals, idx):
    B, N = X_neuron.shape
    out_sc = _make_sc_kernel(B, N, _SC_ROWS)(X_neuron, mean, std, vals, idx)
    w = vals / std
    c = -jnp.sum(vals * mean / std)
    bt = idx // _TW
    coff = idx % _TW
    out_tc = _make_tc_kernel(B, N, _SC_ROWS)(
        bt, X_neuron, w, coff, c[None])
    return jnp.concatenate([out_sc, out_tc])


# speedup vs baseline: 2.1953x; 1.0236x over previous
import functools

import jax
import jax.numpy as jnp
from jax import lax
from jax.experimental import pallas as pl
from jax.experimental.pallas import tpu as pltpu
from jax.experimental.pallas import tpu_sc as plsc

_NC = 2
_NS = 16
_NW = _NC * _NS
_L = 16
_J = 10
_TW = 128
_NBUF = 4
_SC_ROWS = 9216
_BR = 1024


@functools.cache
def _make_sc_kernel(B, N, S):
    b_per_w = S // _NW
    bc = b_per_w // 2
    n_chunks = b_per_w // bc
    grp_per_chunk = bc // _L

    mesh = plsc.VectorSubcoreMesh(
        core_axis_name="c", subcore_axis_name="s",
        num_cores=_NC, num_subcores=_NS)

    @functools.partial(
        pl.kernel,
        out_type=jax.ShapeDtypeStruct((S,), jnp.float32),
        mesh=mesh,
        compiler_params=pltpu.CompilerParams(needs_layout_passes=False),
        scratch_types=[
            pltpu.VMEM((_L,), jnp.float32),
            pltpu.VMEM((_L,), jnp.float32),
            pltpu.VMEM((_L,), jnp.float32),
            pltpu.VMEM((_L,), jnp.int32),
            pltpu.VMEM((_NBUF, bc, _TW), jnp.float32),
            pltpu.VMEM((b_per_w,), jnp.float32),
            pltpu.SemaphoreType.DMA,
            pltpu.SemaphoreType.DMA,
            pltpu.SemaphoreType.DMA,
            pltpu.SemaphoreType.DMA,
        ],
    )
    def sc_kernel(x2d, mean_in, std_in, vals_in, idx_in, out,
                  mean_v, std_v, vals_v, idx_v, ring, outbuf,
                  sem0, sem1, sem2, sem3):
        sems = (sem0, sem1, sem2, sem3)
        wid = lax.axis_index("s") * _NC + lax.axis_index("c")
        base = wid * b_per_w

        cps = [
            pltpu.async_copy(mean_in, mean_v.at[pl.ds(0, _J)], sem0),
            pltpu.async_copy(std_in, std_v.at[pl.ds(0, _J)], sem0),
            pltpu.async_copy(vals_in, vals_v.at[pl.ds(0, _J)], sem0),
            pltpu.async_copy(idx_in, idx_v.at[pl.ds(0, _J)], sem0),
        ]
        for cp in cps:
            cp.wait()

        lane = lax.iota(jnp.int32, _L)
        inb = lane < _J

        def lane_scalar(vec, j):
            return jnp.sum(jnp.where(lane == j, vec, jnp.zeros_like(vec)))

        mean_v16 = mean_v[...]
        std_v16 = std_v[...]
        vals_v16 = vals_v[...]
        iv = idx_v[...]

        col = [lane_scalar(iv, j) for j in range(_J)]
        band = [(col[j] // _TW) * _TW for j in range(_J)]
        coff = [jnp.full((_L,), col[j] % _TW) for j in range(_J)]

        wv = vals_v16 / std_v16
        c = -jnp.sum(jnp.where(inb, vals_v16 * mean_v16 / std_v16, 0.0))
        wj = [lane_scalar(wv, j) for j in range(_J)]

        work = [(ch, j) for ch in range(n_chunks) for j in range(_J)]

        def fire(k):
            ch, j = work[k]
            slot = k % _NBUF
            return pltpu.async_copy(
                x2d.at[pl.ds(base + ch * bc, bc), pl.ds(band[j], _TW)],
                ring.at[slot], sems[slot])

        copies = {}
        for k in range(_NBUF):
            copies[k] = fire(k)

        rows = [jnp.full((_L,), g * _L) + lane for g in range(grp_per_chunk)]
        for ch in range(n_chunks):
            acc = [jnp.full((_L,), c, jnp.float32)
                   for _ in range(grp_per_chunk)]
            for j in range(_J):
                k = ch * _J + j
                slot = k % _NBUF
                copies[k].wait()
                for g in range(grp_per_chunk):
                    vec = plsc.load_gather(ring.at[slot], [rows[g], coff[j]])
                    acc[g] = acc[g] + wj[j] * vec
                nxt = k + _NBUF
                if nxt < len(work):
                    copies[nxt] = fire(nxt)
            for g in range(grp_per_chunk):
                outbuf[pl.ds(ch * bc + g * _L, _L)] = acc[g]

        pltpu.sync_copy(outbuf, out.at[pl.ds(base, b_per_w)])

    return sc_kernel


@functools.cache
def _make_tc_kernel(B, N, S):
    R = B - S

    def tc_body(bt_ref, x_ref, w_ref, coff_ref, c_ref, o_ref, bufs, sems):
        cps = []
        for j in range(_J):
            cstart = pl.multiple_of(bt_ref[j] * _TW, _TW)
            cp = pltpu.make_async_copy(
                x_ref.at[pl.ds(S, R), pl.ds(cstart, _TW)],
                bufs.at[j], sems.at[j])
            cp.start()
            cps.append(cp)
        acc = jnp.full((R,), c_ref[0], jnp.float32)
        iota = lax.broadcasted_iota(jnp.int32, (1, _TW), 1)
        for j in range(_J):
            cps[j].wait()
            onehot = jnp.where(iota == coff_ref[j], w_ref[j], 0.0)[0]
            acc = acc + jnp.dot(bufs[j], onehot,
                                precision=lax.Precision.HIGHEST,
                                preferred_element_type=jnp.float32)
        o_ref[...] = acc

    grid_spec = pltpu.PrefetchScalarGridSpec(
        num_scalar_prefetch=1,
        grid=(1,),
        in_specs=[
            pl.BlockSpec(memory_space=pltpu.MemorySpace.HBM),
            pl.BlockSpec(memory_space=pltpu.SMEM),
            pl.BlockSpec(memory_space=pltpu.SMEM),
            pl.BlockSpec(memory_space=pltpu.SMEM),
        ],
        out_specs=pl.BlockSpec((R,), lambda i, bt: (0,)),
        scratch_shapes=[
            pltpu.VMEM((_J, R, _TW), jnp.float32),
            pltpu.SemaphoreType.DMA((_J,)),
        ],
    )

    return pl.pallas_call(
        tc_body,
        grid_spec=grid_spec,
        out_shape=jax.ShapeDtypeStruct((R,), jnp.float32),
    )


def kernel(X_neuron, mean, std, vals, idx):
    B, N = X_neuron.shape
    out_sc = _make_sc_kernel(B, N, _SC_ROWS)(X_neuron, mean, std, vals, idx)
    w = vals / std
    c = -jnp.sum(vals * mean / std)
    bt = idx // _TW
    coff = idx % _TW
    out_tc = _make_tc_kernel(B, N, _SC_ROWS)(
        bt, X_neuron, w, coff, c[None])
    return jnp.concatenate([out_sc, out_tc])
